# Initial kernel scaffold; baseline (speedup 1.0000x reference)
import jax
import jax.numpy as jnp
from jax.experimental import pallas as pl


def kernel(x, tok_embed, pos_embed):
    raise NotImplementedError("write your pallas kernel here")


# speedup vs baseline: 2.1268x; 2.1268x over previous
import functools

import jax
import jax.numpy as jnp
from jax import lax
from jax.experimental import pallas as pl
from jax.experimental.pallas import tpu as pltpu
from jax.experimental.pallas import tpu_sc as plsc

VOCAB = 128
SEQ = 72
D = 1024
BATCH = 4096

NC = 2
NS = 16
NW = NC * NS
LANES = 16

NTOK = BATCH * SEQ
TOK_PER_W = NTOK // NW
CHUNK = 96
N_CHUNKS = TOK_PER_W // CHUNK


S_BLK = 8


def _fused_body(tok_ref, pos_ref, out_ref):
    out_ref[...] = tok_ref[...][None, :, :] + pos_ref[...][:, None, :]


def _build_fused(tok_embed, pos2d):
    return pl.pallas_call(
        _fused_body,
        grid=(SEQ // S_BLK,),
        in_specs=[
            pl.BlockSpec((VOCAB, D), lambda s: (0, 0)),
            pl.BlockSpec((S_BLK, D), lambda s: (s, 0)),
        ],
        out_specs=pl.BlockSpec((S_BLK, VOCAB, D), lambda s: (s, 0, 0)),
        out_shape=jax.ShapeDtypeStruct((SEQ, VOCAB, D), jnp.float32),
    )(tok_embed, pos2d)


_MESH = plsc.VectorSubcoreMesh(core_axis_name="c", subcore_axis_name="s")


@functools.partial(
    pl.kernel,
    out_type=jax.ShapeDtypeStruct((NTOK, D), jnp.float32),
    mesh=_MESH,
    scratch_types=[
        pltpu.VMEM((CHUNK,), jnp.int32),
        pltpu.VMEM((CHUNK, D), jnp.float32),
        pltpu.SemaphoreType.DMA,
    ],
)
def _gather_kernel(idx_hbm, fused_hbm, out_hbm, idx_v, rows_v, sem):
    wid = lax.axis_index("s") * NC + lax.axis_index("c")
    base = wid * TOK_PER_W

    def step(i, carry):
        off = base + i * CHUNK
        pltpu.sync_copy(idx_hbm.at[pl.ds(off, CHUNK)], idx_v)

        def fuse(j, c):
            p0 = off + j * LANES
            p = p0 + lax.iota(jnp.int32, LANES)
            s = lax.rem(p, SEQ)
            idx_v[pl.ds(j * LANES, LANES)] = (
                s * VOCAB + idx_v[pl.ds(j * LANES, LANES)]
            )
            return c

        lax.fori_loop(0, CHUNK // LANES, fuse, 0, unroll=True)

        pltpu.async_copy(fused_hbm.at[idx_v], rows_v, sem).wait()
        pltpu.sync_copy(rows_v, out_hbm.at[pl.ds(off, CHUNK)])
        return carry

    lax.fori_loop(0, N_CHUNKS, step, 0)


def kernel(x, tok_embed, pos_embed):
    pos2d = pos_embed.reshape(SEQ, D).astype(jnp.float32)
    fused = _build_fused(tok_embed.astype(jnp.float32), pos2d)
    fused_flat = fused.reshape(SEQ * VOCAB, D)
    xflat = x.reshape(NTOK).astype(jnp.int32)
    out = _gather_kernel(xflat, fused_flat)
    return out.reshape(BATCH, SEQ, D)


# speedup vs baseline: 2.3209x; 1.0913x over previous
import functools

import jax
import jax.numpy as jnp
from jax import lax
from jax.experimental import pallas as pl
from jax.experimental.pallas import tpu as pltpu
from jax.experimental.pallas import tpu_sc as plsc

VOCAB = 128
SEQ = 72
D = 1024
BATCH = 4096

NC = 2
NS = 16
NW = NC * NS
LANES = 16

NTOK = BATCH * SEQ
TOK_PER_W = NTOK // NW
CHUNK = 48
N_CHUNKS = TOK_PER_W // CHUNK
NBUF = 2


S_BLK = 8


def _fused_body(tok_ref, pos_ref, out_ref):
    out_ref[...] = tok_ref[...][None, :, :] + pos_ref[...][:, None, :]


def _build_fused(tok_embed, pos2d):
    return pl.pallas_call(
        _fused_body,
        grid=(SEQ // S_BLK,),
        in_specs=[
            pl.BlockSpec((VOCAB, D), lambda s: (0, 0)),
            pl.BlockSpec((S_BLK, D), lambda s: (s, 0)),
        ],
        out_specs=pl.BlockSpec((S_BLK, VOCAB, D), lambda s: (s, 0, 0)),
        out_shape=jax.ShapeDtypeStruct((SEQ, VOCAB, D), jnp.float32),
    )(tok_embed, pos2d)


_MESH = plsc.VectorSubcoreMesh(core_axis_name="c", subcore_axis_name="s")


@functools.partial(
    pl.kernel,
    out_type=jax.ShapeDtypeStruct((NTOK, D), jnp.float32),
    mesh=_MESH,
    scratch_types=[
        pltpu.VMEM((N_CHUNKS, CHUNK), jnp.int32),
        [pltpu.VMEM((CHUNK, D), jnp.float32) for _ in range(NBUF)],
        [pltpu.SemaphoreType.DMA for _ in range(NBUF)],
    ],
)
def _gather_kernel(idx_hbm, fused_hbm, out_hbm, idx_v, rows, sems):
    wid = lax.axis_index("s") * NC + lax.axis_index("c")
    base = wid * TOK_PER_W

    pltpu.sync_copy(idx_hbm.at[pl.ds(wid * N_CHUNKS, N_CHUNKS)], idx_v)

    def fuse(c, carry):
        for l in range(CHUNK // LANES):
            p = base + c * CHUNK + l * LANES + lax.iota(jnp.int32, LANES)
            sl = (c, pl.ds(l * LANES, LANES))
            idx_v[sl] = lax.rem(p, SEQ) * VOCAB + idx_v[sl]
        return carry

    lax.fori_loop(0, N_CHUNKS, fuse, 0)

    def fire(i, b):
        pltpu.async_copy(fused_hbm.at[idx_v.at[i]], rows[b], sems[b])

    def wait(i, b):
        pltpu.make_async_copy(fused_hbm.at[idx_v.at[i]], rows[b], sems[b]).wait()

    def put(i, b):
        pltpu.sync_copy(rows[b], out_hbm.at[pl.ds(base + i * CHUNK, CHUNK)])

    for b in range(NBUF):
        fire(b, b)

    def step(k, carry):
        for b in range(NBUF):
            i = k * NBUF + b
            wait(i, b)
            put(i, b)
            fire(i + NBUF, b)
        return carry

    lax.fori_loop(0, N_CHUNKS // NBUF - 1, step, 0)

    for b in range(NBUF):
        i = N_CHUNKS - NBUF + b
        wait(i, b)
        put(i, b)


def kernel(x, tok_embed, pos_embed):
    pos2d = pos_embed.reshape(SEQ, D).astype(jnp.float32)
    fused = _build_fused(tok_embed.astype(jnp.float32), pos2d)
    fused_flat = fused.reshape(SEQ * VOCAB, D)
    x2d = x.reshape(NW * N_CHUNKS, CHUNK).astype(jnp.int32)
    out = _gather_kernel(x2d, fused_flat)
    return out.reshape(BATCH, SEQ, D)
